# Initial kernel scaffold; baseline (speedup 1.0000x reference)
import jax
import jax.numpy as jnp
from jax.experimental import pallas as pl


def kernel(x_real, x_imag):
    raise NotImplementedError("write your pallas kernel here")


# speedup vs baseline: 2445.3923x; 2445.3923x over previous
import jax
import jax.numpy as jnp
from jax.experimental import pallas as pl

_N = 1 << 20
_B = 1 << 10


def _wht_kernel(x_ref, o_ref):
    r = jax.lax.broadcasted_iota(jnp.int32, (_B, _B), 0)
    c = jax.lax.broadcasted_iota(jnp.int32, (_B, _B), 1)
    k = r & c
    k = k ^ (k >> 1)
    k = k ^ (k >> 2)
    k = k ^ (k >> 4)
    k = k ^ (k >> 8)
    h = (1 - 2 * (k & 1)).astype(jnp.float32)
    x = x_ref[0]
    t = jax.lax.dot(x, h, precision=jax.lax.Precision.HIGHEST,
                    preferred_element_type=jnp.float32)
    o_ref[0] = jax.lax.dot(h, t, precision=jax.lax.Precision.HIGHEST,
                           preferred_element_type=jnp.float32)


def kernel(x_real, x_imag):
    x = jnp.stack([x_real.reshape(_B, _B), x_imag.reshape(_B, _B)])
    y = pl.pallas_call(
        _wht_kernel,
        grid=(2,),
        in_specs=[pl.BlockSpec((1, _B, _B), lambda i: (i, 0, 0))],
        out_specs=pl.BlockSpec((1, _B, _B), lambda i: (i, 0, 0)),
        out_shape=jax.ShapeDtypeStruct((2, _B, _B), jnp.float32),
    )(x)
    return y[0].reshape(_N), y[1].reshape(_N)


# speedup vs baseline: 5175.4915x; 2.1164x over previous
import jax
import jax.numpy as jnp
from jax.experimental import pallas as pl

_N = 1 << 20
_B = 1 << 10


def _wht_kernel(x_ref, o_ref):
    r = jax.lax.broadcasted_iota(jnp.int32, (_B, _B), 0)
    c = jax.lax.broadcasted_iota(jnp.int32, (_B, _B), 1)
    k = r & c
    k = k ^ (k >> 1)
    k = k ^ (k >> 2)
    k = k ^ (k >> 4)
    k = k ^ (k >> 8)
    h = (1 - 2 * (k & 1)).astype(jnp.float32)
    x = x_ref[0]
    t = jax.lax.dot(x, h, preferred_element_type=jnp.float32)
    o_ref[0] = jax.lax.dot(h, t, preferred_element_type=jnp.float32)


def kernel(x_real, x_imag):
    x = jnp.stack([x_real.reshape(_B, _B), x_imag.reshape(_B, _B)])
    y = pl.pallas_call(
        _wht_kernel,
        grid=(2,),
        in_specs=[pl.BlockSpec((1, _B, _B), lambda i: (i, 0, 0))],
        out_specs=pl.BlockSpec((1, _B, _B), lambda i: (i, 0, 0)),
        out_shape=jax.ShapeDtypeStruct((2, _B, _B), jnp.float32),
    )(x)
    return y[0].reshape(_N), y[1].reshape(_N)


# speedup vs baseline: 6564.1368x; 1.2683x over previous
import jax
import jax.numpy as jnp
from jax.experimental import pallas as pl

_N = 1 << 20
_B = 1 << 10


def _wht_kernel(xr_ref, xi_ref, or_ref, oi_ref):
    r = jax.lax.broadcasted_iota(jnp.int32, (_B, _B), 0)
    c = jax.lax.broadcasted_iota(jnp.int32, (_B, _B), 1)
    k = r & c
    k = k ^ (k >> 1)
    k = k ^ (k >> 2)
    k = k ^ (k >> 4)
    k = k ^ (k >> 8)
    h = (1 - 2 * (k & 1)).astype(jnp.bfloat16)
    for x_ref, o_ref in ((xr_ref, or_ref), (xi_ref, oi_ref)):
        x = x_ref[...].astype(jnp.bfloat16)
        t = jax.lax.dot(x, h, preferred_element_type=jnp.float32)
        o_ref[...] = jax.lax.dot(h, t.astype(jnp.bfloat16),
                                 preferred_element_type=jnp.float32)


def kernel(x_real, x_imag):
    yr, yi = pl.pallas_call(
        _wht_kernel,
        out_shape=(jax.ShapeDtypeStruct((_B, _B), jnp.float32),
                   jax.ShapeDtypeStruct((_B, _B), jnp.float32)),
    )(x_real.reshape(_B, _B), x_imag.reshape(_B, _B))
    return yr.reshape(_N), yi.reshape(_N)
